# baseline (speedup 1.0000x reference)
import functools

import jax
import jax.numpy as jnp
from jax import lax
from jax.experimental import pallas as pl
from jax.experimental.pallas import tpu as pltpu
from jax.experimental.pallas import tpu_sc as plsc

_NUM_WORKERS = 32
_NUM_CORES = 2


def _sc_gather_body(b_per_w, ids_a, ids_b, tab_a, tab_b, out_a, out_b,
                    idx_a_v, idx_b_v, rows_a_v, rows_b_v, sem):
    wid = lax.axis_index("s") * _NUM_CORES + lax.axis_index("c")
    base = wid * b_per_w
    pltpu.sync_copy(ids_a.at[pl.ds(base, b_per_w)], idx_a_v)
    cp_a = pltpu.async_copy(tab_a.at[idx_a_v], rows_a_v, sem)
    pltpu.sync_copy(ids_b.at[pl.ds(base, b_per_w)], idx_b_v)
    cp_b = pltpu.async_copy(tab_b.at[idx_b_v], rows_b_v, sem)
    cp_a.wait()
    pltpu.sync_copy(rows_a_v, out_a.at[pl.ds(base, b_per_w)])
    cp_b.wait()
    pltpu.sync_copy(rows_b_v, out_b.at[pl.ds(base, b_per_w)])


def _sc_gather(ids_a, ids_b, tab_a, tab_b):
    b = ids_a.shape[0]
    d = tab_a.shape[1]
    b_per_w = b // _NUM_WORKERS
    mesh = plsc.VectorSubcoreMesh(core_axis_name="c", subcore_axis_name="s")
    f = pl.kernel(
        functools.partial(_sc_gather_body, b_per_w),
        mesh=mesh,
        out_type=[
            jax.ShapeDtypeStruct((b, d), jnp.float32),
            jax.ShapeDtypeStruct((b, d), jnp.float32),
        ],
        scratch_types=[
            pltpu.VMEM((b_per_w,), jnp.int32),
            pltpu.VMEM((b_per_w,), jnp.int32),
            pltpu.VMEM((b_per_w, d), jnp.float32),
            pltpu.VMEM((b_per_w, d), jnp.float32),
            pltpu.SemaphoreType.DMA,
        ],
        compiler_params=pltpu.CompilerParams(use_tc_tiling_on_sc=False),
    )
    return f(ids_a, ids_b, tab_a, tab_b)


def _masked_avg(rows):
    denom = (jnp.sum(rows, axis=1, keepdims=True) != 0).astype(jnp.float32)
    avg = rows / denom
    return jnp.where(jnp.isnan(avg), 0.0, avg)


def _mlp_body(xf_ref, ra_ref, rb_ref, w0t_ref, b0_ref, w1t_ref, b1_ref,
              w2_ref, b2_ref, out_ref):
    avg_a = _masked_avg(ra_ref[...])
    avg_b = _masked_avg(rb_ref[...])
    x = jnp.concatenate([xf_ref[...], avg_a, avg_b], axis=1)
    h = jnp.dot(x, w0t_ref[...], preferred_element_type=jnp.float32)
    h = jnp.maximum(h + b0_ref[...], 0.0)
    h = jnp.dot(h, w1t_ref[...], preferred_element_type=jnp.float32)
    h = jnp.maximum(h + b1_ref[...], 0.0)
    o = jnp.sum(h * w2_ref[...], axis=1, keepdims=True) + b2_ref[...]
    out_ref[...] = o


def _mlp(xf, rows_a, rows_b, w0, b0, w1, b1, w2, b2):
    b, d_float = xf.shape
    d = rows_a.shape[1]
    h0 = w0.shape[0]
    h1 = w1.shape[0]
    blk = 1024
    grid = (b // blk,)
    const = lambda i: (0, 0)
    return pl.pallas_call(
        _mlp_body,
        grid=grid,
        in_specs=[
            pl.BlockSpec((blk, d_float), lambda i: (i, 0)),
            pl.BlockSpec((blk, d), lambda i: (i, 0)),
            pl.BlockSpec((blk, d), lambda i: (i, 0)),
            pl.BlockSpec((d_float + 2 * d, h0), const),
            pl.BlockSpec((1, h0), const),
            pl.BlockSpec((h0, h1), const),
            pl.BlockSpec((1, h1), const),
            pl.BlockSpec((1, h1), const),
            pl.BlockSpec((1, 1), const),
        ],
        out_specs=pl.BlockSpec((blk, 1), lambda i: (i, 0)),
        out_shape=jax.ShapeDtypeStruct((b, 1), jnp.float32),
    )(xf, rows_a, rows_b, w0.T, b0.reshape(1, -1), w1.T, b1.reshape(1, -1),
      w2, b2.reshape(1, 1))


def kernel(X_float, X_id_list, X_id_list_idxs, Emb_a, Emb_b,
           W0, b0, W1, b1, W2, b2):
    idxs = X_id_list_idxs[0]
    ids = X_id_list.astype(jnp.int32)
    ids_a = lax.dynamic_slice_in_dim(ids, idxs[0], 1, axis=1).reshape(-1)
    ids_b = lax.dynamic_slice_in_dim(ids, idxs[2], 1, axis=1).reshape(-1)
    rows_a, rows_b = _sc_gather(ids_a, ids_b, Emb_a, Emb_b)
    return _mlp(X_float, rows_a, rows_b, W0, b0, W1, b1, W2, b2)


# speedup vs baseline: 1.5157x; 1.5157x over previous
import functools

import jax
import jax.numpy as jnp
from jax import lax
from jax.experimental import pallas as pl
from jax.experimental.pallas import tpu as pltpu
from jax.experimental.pallas import tpu_sc as plsc

_NUM_WORKERS = 32
_NUM_CORES = 2
_CHUNK = 16


def _gather_one_table(b_per_w, base, ids, tab, out, idx_v, rows_v, sem):
    pltpu.sync_copy(ids.at[pl.ds(base, b_per_w)], idx_v)
    n_chunks = b_per_w // _CHUNK

    def fire(c):
        cbase = c * _CHUNK
        vec = idx_v[pl.ds(cbase, _CHUNK)]
        for j in range(_CHUNK):
            idx = vec[j]
            pltpu.async_copy(
                tab.at[pl.ds(idx, 1)], rows_v.at[pl.ds(cbase + j, 1)], sem
            )

    def drain(c):
        pltpu.make_async_copy(
            tab.at[pl.ds(0, _CHUNK)], rows_v.at[pl.ds(c * _CHUNK, _CHUNK)], sem
        ).wait()

    fire(0)

    def body(c, carry):
        fire(c)
        drain(c - 1)
        return carry

    lax.fori_loop(1, n_chunks, body, 0)
    drain(n_chunks - 1)
    pltpu.sync_copy(rows_v, out.at[pl.ds(base, b_per_w)])


def _sc_gather_body(b_per_w, ids_a, ids_b, tab_a, tab_b, out_a, out_b,
                    idx_v, rows_v, sem):
    wid = lax.axis_index("s") * _NUM_CORES + lax.axis_index("c")
    base = wid * b_per_w
    _gather_one_table(b_per_w, base, ids_a, tab_a, out_a, idx_v, rows_v, sem)
    _gather_one_table(b_per_w, base, ids_b, tab_b, out_b, idx_v, rows_v, sem)


def _sc_gather(ids_a, ids_b, tab_a, tab_b):
    b = ids_a.shape[0]
    d = tab_a.shape[1]
    b_per_w = b // _NUM_WORKERS
    mesh = plsc.VectorSubcoreMesh(core_axis_name="c", subcore_axis_name="s")
    f = pl.kernel(
        functools.partial(_sc_gather_body, b_per_w),
        mesh=mesh,
        out_type=[
            jax.ShapeDtypeStruct((b, d), jnp.float32),
            jax.ShapeDtypeStruct((b, d), jnp.float32),
        ],
        scratch_types=[
            pltpu.VMEM((b_per_w,), jnp.int32),
            pltpu.VMEM((b_per_w, d), jnp.float32),
            pltpu.SemaphoreType.DMA,
        ],
    )
    return f(ids_a, ids_b, tab_a, tab_b)


def _masked_avg(rows):
    denom = (jnp.sum(rows, axis=1, keepdims=True) != 0).astype(jnp.float32)
    avg = rows / denom
    return jnp.where(jnp.isnan(avg), 0.0, avg)


def _mlp_body(xf_ref, ra_ref, rb_ref, w0t_ref, b0_ref, w1t_ref, b1_ref,
              w2_ref, b2_ref, out_ref):
    avg_a = _masked_avg(ra_ref[...])
    avg_b = _masked_avg(rb_ref[...])
    x = jnp.concatenate([xf_ref[...], avg_a, avg_b], axis=1)
    h = jnp.dot(x, w0t_ref[...], preferred_element_type=jnp.float32)
    h = jnp.maximum(h + b0_ref[...], 0.0)
    h = jnp.dot(h, w1t_ref[...], preferred_element_type=jnp.float32)
    h = jnp.maximum(h + b1_ref[...], 0.0)
    o = jnp.sum(h * w2_ref[...], axis=1, keepdims=True) + b2_ref[...]
    out_ref[...] = o


def _mlp(xf, rows_a, rows_b, w0, b0, w1, b1, w2, b2):
    b, d_float = xf.shape
    d = rows_a.shape[1]
    h0 = w0.shape[0]
    h1 = w1.shape[0]
    blk = 1024
    grid = (b // blk,)
    const = lambda i: (0, 0)
    return pl.pallas_call(
        _mlp_body,
        grid=grid,
        in_specs=[
            pl.BlockSpec((blk, d_float), lambda i: (i, 0)),
            pl.BlockSpec((blk, d), lambda i: (i, 0)),
            pl.BlockSpec((blk, d), lambda i: (i, 0)),
            pl.BlockSpec((d_float + 2 * d, h0), const),
            pl.BlockSpec((1, h0), const),
            pl.BlockSpec((h0, h1), const),
            pl.BlockSpec((1, h1), const),
            pl.BlockSpec((1, h1), const),
            pl.BlockSpec((1, 1), const),
        ],
        out_specs=pl.BlockSpec((blk, 1), lambda i: (i, 0)),
        out_shape=jax.ShapeDtypeStruct((b, 1), jnp.float32),
    )(xf, rows_a, rows_b, w0.T, b0.reshape(1, -1), w1.T, b1.reshape(1, -1),
      w2, b2.reshape(1, 1))


def kernel(X_float, X_id_list, X_id_list_idxs, Emb_a, Emb_b,
           W0, b0, W1, b1, W2, b2):
    idxs = X_id_list_idxs[0]
    ids = X_id_list.astype(jnp.int32)
    ids_a = lax.dynamic_slice_in_dim(ids, idxs[0], 1, axis=1).reshape(-1)
    ids_b = lax.dynamic_slice_in_dim(ids, idxs[2], 1, axis=1).reshape(-1)
    rows_a, rows_b = _sc_gather(ids_a, ids_b, Emb_a, Emb_b)
    return _mlp(X_float, rows_a, rows_b, W0, b0, W1, b1, W2, b2)
